# Initial kernel scaffold; baseline (speedup 1.0000x reference)
import jax
import jax.numpy as jnp
from jax.experimental import pallas as pl


def kernel(h, W_sel, b_sel, W1, W2):
    raise NotImplementedError("write your pallas kernel here")


# speedup vs baseline: 2.2162x; 2.2162x over previous
import functools

import jax
import jax.numpy as jnp
from jax.experimental import pallas as pl
from jax.experimental.pallas import tpu as pltpu

B = 4
S = 2048
DIM = 1024
NA = 3
TS = 512


def _router_kernel(h_ref, wsel_ref, bsel_ref, idx_ref, act_ref):
    pooled = jnp.sum(h_ref[0], axis=0) * (1.0 / S)
    best = jnp.sum(wsel_ref[0] * pooled) + bsel_ref[0]
    best_i = jnp.int32(0)
    for j in range(1, NA + 1):
        lj = jnp.sum(wsel_ref[j] * pooled) + bsel_ref[j]
        take = lj > best
        best_i = jnp.where(take, jnp.int32(j), best_i)
        best = jnp.maximum(lj, best)
    active = (best_i >= 1).astype(jnp.int32)
    idx_ref[0, 0, 0] = jnp.where(best_i >= 1, best_i - 1, 0)
    act_ref[0, 0, 0] = active


def _mlp_kernel(idx_ref, act_ref, h_ref, w1_ref, w2_ref, o_ref):
    del idx_ref
    b = pl.program_id(0)

    @pl.when(act_ref[b, 0, 0] == 1)
    def _compute():
        x = h_ref[0].astype(jnp.bfloat16)
        hid = jnp.dot(x, w1_ref[0], preferred_element_type=jnp.float32)
        hid = 0.5 * hid * (1.0 + jax.lax.erf(hid * 0.7071067811865476))
        delta = jnp.dot(hid.astype(jnp.bfloat16), w2_ref[0],
                        preferred_element_type=jnp.float32)
        o_ref[0] = h_ref[0] + delta

    @pl.when(act_ref[b, 0, 0] == 0)
    def _copy():
        o_ref[0] = h_ref[0]


@jax.jit
def kernel(h, W_sel, b_sel, W1, W2):
    idx, act = pl.pallas_call(
        _router_kernel,
        grid=(B,),
        in_specs=[
            pl.BlockSpec((1, S, DIM), lambda b: (b, 0, 0)),
            pl.BlockSpec((NA + 1, DIM), lambda b: (0, 0)),
            pl.BlockSpec(memory_space=pltpu.SMEM),
        ],
        out_specs=[
            pl.BlockSpec((1, 1, 1), lambda b: (b, 0, 0), memory_space=pltpu.SMEM),
            pl.BlockSpec((1, 1, 1), lambda b: (b, 0, 0), memory_space=pltpu.SMEM),
        ],
        out_shape=[
            jax.ShapeDtypeStruct((B, 1, 1), jnp.int32),
            jax.ShapeDtypeStruct((B, 1, 1), jnp.int32),
        ],
        compiler_params=pltpu.CompilerParams(
            dimension_semantics=(pltpu.ARBITRARY,),
        ),
    )(h, W_sel, b_sel)

    w1b = W1.astype(jnp.bfloat16)
    w2b = W2.astype(jnp.bfloat16)

    grid_spec = pltpu.PrefetchScalarGridSpec(
        num_scalar_prefetch=2,
        grid=(B, S // TS),
        in_specs=[
            pl.BlockSpec((1, TS, DIM), lambda b, s, idx, act: (b, s, 0)),
            pl.BlockSpec((1, DIM, DIM), lambda b, s, idx, act: (idx[b, 0, 0], 0, 0)),
            pl.BlockSpec((1, DIM, DIM), lambda b, s, idx, act: (idx[b, 0, 0], 0, 0)),
        ],
        out_specs=pl.BlockSpec((1, TS, DIM), lambda b, s, idx, act: (b, s, 0)),
    )
    out = pl.pallas_call(
        _mlp_kernel,
        grid_spec=grid_spec,
        out_shape=jax.ShapeDtypeStruct((B, S, DIM), jnp.float32),
        compiler_params=pltpu.CompilerParams(
            dimension_semantics=(pltpu.PARALLEL, pltpu.ARBITRARY),
        ),
    )(idx, act, h, w1b, w2b)
    return out


# speedup vs baseline: 2.7853x; 1.2568x over previous
import jax
import jax.numpy as jnp
from jax.experimental import pallas as pl
from jax.experimental.pallas import tpu as pltpu

B = 4
S = 2048
DIM = 1024
NA = 3
TS = 512


def _router_kernel(h_ref, wsel_ref, bsel_ref, idx_ref, act_ref, cast_ref,
                   last_ref):
    b = pl.program_id(0)
    pooled = jnp.sum(h_ref[0], axis=0) * (1.0 / S)
    best = jnp.sum(wsel_ref[0] * pooled) + bsel_ref[0]
    best_i = jnp.int32(0)
    for j in range(1, NA + 1):
        lj = jnp.sum(wsel_ref[j] * pooled) + bsel_ref[j]
        take = lj > best
        best_i = jnp.where(take, jnp.int32(j), best_i)
        best = jnp.maximum(lj, best)
    active = best_i >= 1
    sel = best_i - 1
    prev = jnp.where(b == 0, 0, last_ref[0])
    have = jnp.where(b == 0, 0, last_ref[1])
    need = jnp.logical_and(active, jnp.logical_or(have == 0, sel != prev))
    eff = jnp.where(active, sel, prev)
    last_ref[0] = eff
    last_ref[1] = jnp.where(active, 1, have)
    idx_ref[0, 0, 0] = eff
    act_ref[0, 0, 0] = active.astype(jnp.int32)
    cast_ref[0, 0, 0] = need.astype(jnp.int32)


def _mlp_kernel(idx_ref, act_ref, cast_ref, h_ref, w1_ref, w2_ref, o_ref,
                w1b_ref, w2b_ref):
    del idx_ref
    b = pl.program_id(0)
    s = pl.program_id(1)

    @pl.when(jnp.logical_and(s == 0, cast_ref[b, 0, 0] == 1))
    def _cast():
        w1b_ref[...] = w1_ref[0].astype(jnp.bfloat16)
        w2b_ref[...] = w2_ref[0].astype(jnp.bfloat16)

    @pl.when(act_ref[b, 0, 0] == 1)
    def _compute():
        x = h_ref[0].astype(jnp.bfloat16)
        hid = jnp.dot(x, w1b_ref[...], preferred_element_type=jnp.float32)
        hid = 0.5 * hid * (1.0 + jax.lax.erf(hid * 0.7071067811865476))
        delta = jnp.dot(hid.astype(jnp.bfloat16), w2b_ref[...],
                        preferred_element_type=jnp.float32)
        o_ref[0] = h_ref[0] + delta

    @pl.when(act_ref[b, 0, 0] == 0)
    def _copy():
        o_ref[0] = h_ref[0]


@jax.jit
def kernel(h, W_sel, b_sel, W1, W2):
    idx, act, cast = pl.pallas_call(
        _router_kernel,
        grid=(B,),
        in_specs=[
            pl.BlockSpec((1, S, DIM), lambda b: (b, 0, 0)),
            pl.BlockSpec((NA + 1, DIM), lambda b: (0, 0)),
            pl.BlockSpec(memory_space=pltpu.SMEM),
        ],
        out_specs=[
            pl.BlockSpec((1, 1, 1), lambda b: (b, 0, 0), memory_space=pltpu.SMEM),
            pl.BlockSpec((1, 1, 1), lambda b: (b, 0, 0), memory_space=pltpu.SMEM),
            pl.BlockSpec((1, 1, 1), lambda b: (b, 0, 0), memory_space=pltpu.SMEM),
        ],
        out_shape=[
            jax.ShapeDtypeStruct((B, 1, 1), jnp.int32),
            jax.ShapeDtypeStruct((B, 1, 1), jnp.int32),
            jax.ShapeDtypeStruct((B, 1, 1), jnp.int32),
        ],
        scratch_shapes=[pltpu.SMEM((2,), jnp.int32)],
        compiler_params=pltpu.CompilerParams(
            dimension_semantics=(pltpu.ARBITRARY,),
        ),
    )(h, W_sel, b_sel)

    grid_spec = pltpu.PrefetchScalarGridSpec(
        num_scalar_prefetch=3,
        grid=(B, S // TS),
        in_specs=[
            pl.BlockSpec((1, TS, DIM), lambda b, s, idx, act, cst: (b, s, 0)),
            pl.BlockSpec((1, DIM, DIM),
                         lambda b, s, idx, act, cst: (idx[b, 0, 0], 0, 0)),
            pl.BlockSpec((1, DIM, DIM),
                         lambda b, s, idx, act, cst: (idx[b, 0, 0], 0, 0)),
        ],
        out_specs=pl.BlockSpec((1, TS, DIM),
                               lambda b, s, idx, act, cst: (b, s, 0)),
        scratch_shapes=[
            pltpu.VMEM((DIM, DIM), jnp.bfloat16),
            pltpu.VMEM((DIM, DIM), jnp.bfloat16),
        ],
    )
    out = pl.pallas_call(
        _mlp_kernel,
        grid_spec=grid_spec,
        out_shape=jax.ShapeDtypeStruct((B, S, DIM), jnp.float32),
        compiler_params=pltpu.CompilerParams(
            dimension_semantics=(pltpu.ARBITRARY, pltpu.ARBITRARY),
        ),
    )(idx, act, cast, h, W1, W2)
    return out


# speedup vs baseline: 3.1536x; 1.1322x over previous
import jax
import jax.numpy as jnp
from jax.experimental import pallas as pl
from jax.experimental.pallas import tpu as pltpu

B = 4
S = 2048
DIM = 1024
NA = 3


def _fused_kernel(h_ref, wsel_ref, bsel_ref, w1_hbm, w2_hbm, o_ref,
                  w1f_ref, w2f_ref, w1b_ref, w2b_ref, last_ref, sem1, sem2):
    b = pl.program_id(0)
    pooled = jnp.sum(h_ref[0], axis=0) * (1.0 / S)
    best = jnp.sum(wsel_ref[0] * pooled) + bsel_ref[0]
    best_i = jnp.int32(0)
    for j in range(1, NA + 1):
        lj = jnp.sum(wsel_ref[j] * pooled) + bsel_ref[j]
        take = lj > best
        best_i = jnp.where(take, jnp.int32(j), best_i)
        best = jnp.maximum(lj, best)
    active = best_i >= 1
    sel = best_i - 1

    resident = jnp.where(b == 0, jnp.int32(-1), last_ref[0])
    need = jnp.logical_and(active, sel != resident)
    last_ref[0] = jnp.where(active, sel, resident)

    @pl.when(need)
    def _fetch():
        cp1 = pltpu.make_async_copy(w1_hbm.at[sel], w1f_ref, sem1)
        cp1.start()
        cp2 = pltpu.make_async_copy(w2_hbm.at[sel], w2f_ref, sem2)
        cp2.start()
        cp1.wait()
        w1b_ref[...] = w1f_ref[...].astype(jnp.bfloat16)
        cp2.wait()
        w2b_ref[...] = w2f_ref[...].astype(jnp.bfloat16)

    @pl.when(active)
    def _mlp():
        x = h_ref[0].astype(jnp.bfloat16)
        hid = jnp.dot(x, w1b_ref[...], preferred_element_type=jnp.float32)
        hid = 0.5 * hid * (1.0 + jax.lax.erf(hid * 0.7071067811865476))
        delta = jnp.dot(hid.astype(jnp.bfloat16), w2b_ref[...],
                        preferred_element_type=jnp.float32)
        o_ref[0] = h_ref[0] + delta

    @pl.when(jnp.logical_not(active))
    def _copy():
        o_ref[0] = h_ref[0]


@jax.jit
def kernel(h, W_sel, b_sel, W1, W2):
    out = pl.pallas_call(
        _fused_kernel,
        grid=(B,),
        in_specs=[
            pl.BlockSpec((1, S, DIM), lambda b: (b, 0, 0)),
            pl.BlockSpec((NA + 1, DIM), lambda b: (0, 0)),
            pl.BlockSpec(memory_space=pltpu.SMEM),
            pl.BlockSpec(memory_space=pltpu.HBM),
            pl.BlockSpec(memory_space=pltpu.HBM),
        ],
        out_specs=pl.BlockSpec((1, S, DIM), lambda b: (b, 0, 0)),
        out_shape=jax.ShapeDtypeStruct((B, S, DIM), jnp.float32),
        scratch_shapes=[
            pltpu.VMEM((DIM, DIM), jnp.float32),
            pltpu.VMEM((DIM, DIM), jnp.float32),
            pltpu.VMEM((DIM, DIM), jnp.bfloat16),
            pltpu.VMEM((DIM, DIM), jnp.bfloat16),
            pltpu.SMEM((1,), jnp.int32),
            pltpu.SemaphoreType.DMA,
            pltpu.SemaphoreType.DMA,
        ],
        compiler_params=pltpu.CompilerParams(
            dimension_semantics=(pltpu.ARBITRARY,),
        ),
    )(h, W_sel, b_sel, W1, W2)
    return out


# speedup vs baseline: 3.3013x; 1.0469x over previous
import jax
import jax.numpy as jnp
from jax.experimental import pallas as pl
from jax.experimental.pallas import tpu as pltpu

B = 4
S = 2048
DIM = 1024
NA = 3
CH = 512


def _fused_kernel(h_ref, wsel_ref, bsel_ref, w1_hbm, w2_hbm, o_ref,
                  w1f_ref, w2f_ref, w1b_ref, w2b_ref, last_ref, sem1, sem2):
    b = pl.program_id(0)
    pooled = jnp.sum(h_ref[0], axis=0) * (1.0 / S)
    best = jnp.sum(wsel_ref[0] * pooled) + bsel_ref[0]
    best_i = jnp.int32(0)
    for j in range(1, NA + 1):
        lj = jnp.sum(wsel_ref[j] * pooled) + bsel_ref[j]
        take = lj > best
        best_i = jnp.where(take, jnp.int32(j), best_i)
        best = jnp.maximum(lj, best)
    active = best_i >= 1
    sel = best_i - 1

    resident = jnp.where(b == 0, jnp.int32(-1), last_ref[0])
    need = jnp.logical_and(active, sel != resident)
    last_ref[0] = jnp.where(active, sel, resident)

    @pl.when(need)
    def _fetch():
        cp1 = pltpu.make_async_copy(w1_hbm.at[sel], w1f_ref, sem1)
        cp1.start()
        cp2 = pltpu.make_async_copy(w2_hbm.at[sel], w2f_ref, sem2)
        cp2.start()
        cp1.wait()
        w1b_ref[...] = w1f_ref[...].astype(jnp.bfloat16)
        cp2.wait()
        w2b_ref[...] = w2f_ref[...].astype(jnp.bfloat16)

    @pl.when(active)
    def _mlp():
        for c in range(S // CH):
            rows = pl.ds(c * CH, CH)
            x = h_ref[0, rows, :].astype(jnp.bfloat16)
            hid = jnp.dot(x, w1b_ref[...], preferred_element_type=jnp.float32)
            hid = 0.5 * hid * (1.0 + jax.lax.erf(hid * 0.7071067811865476))
            delta = jnp.dot(hid.astype(jnp.bfloat16), w2b_ref[...],
                            preferred_element_type=jnp.float32)
            o_ref[0, rows, :] = h_ref[0, rows, :] + delta

    @pl.when(jnp.logical_not(active))
    def _copy():
        o_ref[0] = h_ref[0]


@jax.jit
def kernel(h, W_sel, b_sel, W1, W2):
    out = pl.pallas_call(
        _fused_kernel,
        grid=(B,),
        in_specs=[
            pl.BlockSpec((1, S, DIM), lambda b: (b, 0, 0)),
            pl.BlockSpec((NA + 1, DIM), lambda b: (0, 0)),
            pl.BlockSpec(memory_space=pltpu.SMEM),
            pl.BlockSpec(memory_space=pltpu.HBM),
            pl.BlockSpec(memory_space=pltpu.HBM),
        ],
        out_specs=pl.BlockSpec((1, S, DIM), lambda b: (b, 0, 0)),
        out_shape=jax.ShapeDtypeStruct((B, S, DIM), jnp.float32),
        scratch_shapes=[
            pltpu.VMEM((DIM, DIM), jnp.float32),
            pltpu.VMEM((DIM, DIM), jnp.float32),
            pltpu.VMEM((DIM, DIM), jnp.bfloat16),
            pltpu.VMEM((DIM, DIM), jnp.bfloat16),
            pltpu.SMEM((1,), jnp.int32),
            pltpu.SemaphoreType.DMA,
            pltpu.SemaphoreType.DMA,
        ],
        compiler_params=pltpu.CompilerParams(
            dimension_semantics=(pltpu.ARBITRARY,),
        ),
    )(h, W_sel, b_sel, W1, W2)
    return out


# speedup vs baseline: 3.4608x; 1.0483x over previous
import jax
import jax.numpy as jnp
from jax.experimental import pallas as pl
from jax.experimental.pallas import tpu as pltpu

B = 4
S = 2048
DIM = 1024
NA = 3
CH = 512

_ACT = 0
_RES = 1
_PEND = 2


def _fused_kernel(h_ref, wsel_ref, bsel_ref, w1_hbm, w2_hbm, o_ref,
                  w1f_ref, w2f_ref, w1b_ref, w2b_ref, xb_ref, state_ref,
                  sem1, sem2):
    b = pl.program_id(0)
    prev_act = jnp.where(b == 0, 0, state_ref[_ACT])
    pending = jnp.where(b == 0, 0, state_ref[_PEND])

    @pl.when(jnp.logical_and(b > 0, prev_act == 1))
    def _mlp():
        @pl.when(pending == 1)
        def _land_weights():
            pltpu.make_async_copy(w1_hbm.at[0], w1f_ref, sem1).wait()
            pltpu.make_async_copy(w2_hbm.at[0], w2f_ref, sem2).wait()
            w1b_ref[...] = w1f_ref[...].astype(jnp.bfloat16)
            w2b_ref[...] = w2f_ref[...].astype(jnp.bfloat16)

        slot = jax.lax.rem(b - 1, 2)
        for c in range(S // CH):
            rows = pl.ds(c * CH, CH)
            x = xb_ref[slot, rows, :]
            hid = jnp.dot(x, w1b_ref[...], preferred_element_type=jnp.float32)
            hid = 0.5 * hid * (1.0 + jax.lax.erf(hid * 0.7071067811865476))
            delta = jnp.dot(hid.astype(jnp.bfloat16), w2b_ref[...],
                            preferred_element_type=jnp.float32)
            o_ref[0, rows, :] = (
                xb_ref[slot, rows, :].astype(jnp.float32) + delta)

    @pl.when(jnp.logical_and(b > 0, prev_act == 0))
    def _copy():
        slot = jax.lax.rem(b - 1, 2)
        o_ref[0] = xb_ref[slot].astype(jnp.float32)

    @pl.when(b < B)
    def _route():
        pooled = jnp.sum(h_ref[0], axis=0) * (1.0 / S)
        best = jnp.sum(wsel_ref[0] * pooled) + bsel_ref[0]
        best_i = jnp.int32(0)
        for j in range(1, NA + 1):
            lj = jnp.sum(wsel_ref[j] * pooled) + bsel_ref[j]
            take = lj > best
            best_i = jnp.where(take, jnp.int32(j), best_i)
            best = jnp.maximum(lj, best)
        active = best_i >= 1
        sel = best_i - 1

        resident = jnp.where(b == 0, jnp.int32(-1), state_ref[_RES])
        need = jnp.logical_and(active, sel != resident)

        @pl.when(need)
        def _fetch():
            pltpu.make_async_copy(w1_hbm.at[sel], w1f_ref, sem1).start()
            pltpu.make_async_copy(w2_hbm.at[sel], w2f_ref, sem2).start()

        state_ref[_ACT] = active.astype(jnp.int32)
        state_ref[_RES] = jnp.where(active, sel, resident)
        state_ref[_PEND] = need.astype(jnp.int32)

        xb_ref[jax.lax.rem(b, 2)] = h_ref[0].astype(jnp.bfloat16)


@jax.jit
def kernel(h, W_sel, b_sel, W1, W2):
    out = pl.pallas_call(
        _fused_kernel,
        grid=(B + 1,),
        in_specs=[
            pl.BlockSpec((1, S, DIM), lambda b: (jnp.minimum(b, B - 1), 0, 0)),
            pl.BlockSpec((NA + 1, DIM), lambda b: (0, 0)),
            pl.BlockSpec(memory_space=pltpu.SMEM),
            pl.BlockSpec(memory_space=pltpu.HBM),
            pl.BlockSpec(memory_space=pltpu.HBM),
        ],
        out_specs=pl.BlockSpec((1, S, DIM), lambda b: (jnp.maximum(b - 1, 0), 0, 0)),
        out_shape=jax.ShapeDtypeStruct((B, S, DIM), jnp.float32),
        scratch_shapes=[
            pltpu.VMEM((DIM, DIM), jnp.float32),
            pltpu.VMEM((DIM, DIM), jnp.float32),
            pltpu.VMEM((DIM, DIM), jnp.bfloat16),
            pltpu.VMEM((DIM, DIM), jnp.bfloat16),
            pltpu.VMEM((2, S, DIM), jnp.bfloat16),
            pltpu.SMEM((3,), jnp.int32),
            pltpu.SemaphoreType.DMA,
            pltpu.SemaphoreType.DMA,
        ],
        compiler_params=pltpu.CompilerParams(
            dimension_semantics=(pltpu.ARBITRARY,),
        ),
    )(h, W_sel, b_sel, W1, W2)
    return out


# speedup vs baseline: 3.5061x; 1.0131x over previous
import jax
import jax.numpy as jnp
from jax.experimental import pallas as pl
from jax.experimental.pallas import tpu as pltpu

B = 4
S = 2048
DIM = 1024
NA = 3
CH = 512

_ACT = 0
_RES = 1
_PEND = 2


def _fused_kernel(h_ref, wsel_ref, bsel_ref, w1_hbm, w2_hbm, o_ref,
                  w1f_ref, w2f_ref, w1b_ref, w2b_ref, xb_ref, state_ref,
                  sem1, sem2):
    b = pl.program_id(0)
    prev_act = jnp.where(b == 0, 0, state_ref[_ACT])
    pending = jnp.where(b == 0, 0, state_ref[_PEND])

    def _route():
        hv = h_ref[0]
        pooled = jnp.sum(hv, axis=0) * (1.0 / S)
        best = jnp.sum(wsel_ref[0] * pooled) + bsel_ref[0]
        best_i = jnp.int32(0)
        for j in range(1, NA + 1):
            lj = jnp.sum(wsel_ref[j] * pooled) + bsel_ref[j]
            take = lj > best
            best_i = jnp.where(take, jnp.int32(j), best_i)
            best = jnp.maximum(lj, best)
        active = best_i >= 1
        sel = best_i - 1

        resident = jnp.where(b == 0, jnp.int32(-1), state_ref[_RES])
        need = jnp.logical_and(active, sel != resident)

        @pl.when(jnp.logical_and(b < B, need))
        def _fetch():
            pltpu.make_async_copy(w1_hbm.at[sel], w1f_ref, sem1).start()
            pltpu.make_async_copy(w2_hbm.at[sel], w2f_ref, sem2).start()

        @pl.when(b < B)
        def _commit():
            state_ref[_ACT] = active.astype(jnp.int32)
            state_ref[_RES] = jnp.where(active, sel, resident)
            state_ref[_PEND] = need.astype(jnp.int32)

        xb_ref[jax.lax.rem(b, 2)] = hv.astype(jnp.bfloat16)

    mlp_pred = jnp.logical_and(b > 0, prev_act == 1)

    @pl.when(mlp_pred)
    def _mlp():
        @pl.when(pending == 1)
        def _land_weights():
            pltpu.make_async_copy(w1_hbm.at[0], w1f_ref, sem1).wait()
            pltpu.make_async_copy(w2_hbm.at[0], w2f_ref, sem2).wait()
            w1b_ref[...] = w1f_ref[...].astype(jnp.bfloat16)
            w2b_ref[...] = w2f_ref[...].astype(jnp.bfloat16)

        slot = jax.lax.rem(b - 1, 2)
        for c in range(S // CH):
            rows = pl.ds(c * CH, CH)
            x = xb_ref[slot, rows, :]
            hid = jnp.dot(x, w1b_ref[...], preferred_element_type=jnp.float32)
            hid = 0.5 * hid * (1.0 + jax.lax.erf(hid * 0.7071067811865476))
            delta = jnp.dot(hid.astype(jnp.bfloat16), w2b_ref[...],
                            preferred_element_type=jnp.float32)
            o_ref[0, rows, :] = x.astype(jnp.float32) + delta
        _route()

    @pl.when(jnp.logical_not(mlp_pred))
    def _copy_or_first():
        @pl.when(jnp.logical_and(b > 0, prev_act == 0))
        def _copy():
            o_ref[0] = xb_ref[jax.lax.rem(b - 1, 2)].astype(jnp.float32)
        _route()


@jax.jit
def kernel(h, W_sel, b_sel, W1, W2):
    out = pl.pallas_call(
        _fused_kernel,
        grid=(B + 1,),
        in_specs=[
            pl.BlockSpec((1, S, DIM), lambda b: (jnp.minimum(b, B - 1), 0, 0)),
            pl.BlockSpec((NA + 1, DIM), lambda b: (0, 0)),
            pl.BlockSpec(memory_space=pltpu.SMEM),
            pl.BlockSpec(memory_space=pltpu.HBM),
            pl.BlockSpec(memory_space=pltpu.HBM),
        ],
        out_specs=pl.BlockSpec((1, S, DIM), lambda b: (jnp.maximum(b - 1, 0), 0, 0)),
        out_shape=jax.ShapeDtypeStruct((B, S, DIM), jnp.float32),
        scratch_shapes=[
            pltpu.VMEM((DIM, DIM), jnp.float32),
            pltpu.VMEM((DIM, DIM), jnp.float32),
            pltpu.VMEM((DIM, DIM), jnp.bfloat16),
            pltpu.VMEM((DIM, DIM), jnp.bfloat16),
            pltpu.VMEM((2, S, DIM), jnp.bfloat16),
            pltpu.SMEM((3,), jnp.int32),
            pltpu.SemaphoreType.DMA,
            pltpu.SemaphoreType.DMA,
        ],
        compiler_params=pltpu.CompilerParams(
            dimension_semantics=(pltpu.ARBITRARY,),
        ),
    )(h, W_sel, b_sel, W1, W2)
    return out
